# baseline (device time: 94574 ns/iter reference)
import jax
import jax.numpy as jnp
from jax import lax
from jax.experimental import pallas as pl
from jax.experimental.pallas import tpu as pltpu

N_DEV = 4
B = 2
SQ = 256
SKV_PER = 256
SKV = SKV_PER * N_DEV
HQ = 4
DH = 64
WINDOW = 128


def kernel(x, Wq, K_ext, V_ext, Wo):
    def body(x_ref, wq_ref, k_ref, v_ref, wo_ref, out_ref,
             kfull, vfull, kcomm, vcomm,
             ksend_sems, krecv_sems, vsend_sems, vrecv_sems):
        my = lax.axis_index("i")
        left = (my - 1) % N_DEV
        right = (my + 1) % N_DEV

        barrier_sem = pltpu.get_barrier_semaphore()
        for nbr in (left, right):
            pl.semaphore_signal(
                barrier_sem, inc=1,
                device_id=(nbr,), device_id_type=pl.DeviceIdType.MESH,
            )
        pl.semaphore_wait(barrier_sem, 2)

        kfull[:, pl.ds(my * SKV_PER, SKV_PER), :, :] = k_ref[...]
        vfull[:, pl.ds(my * SKV_PER, SKV_PER), :, :] = v_ref[...]
        kcomm[0] = k_ref[...]
        vcomm[0] = v_ref[...]

        for h in range(N_DEV - 1):
            send_slot = h % 2
            recv_slot = (h + 1) % 2
            krdma = pltpu.make_async_remote_copy(
                src_ref=kcomm.at[send_slot],
                dst_ref=kcomm.at[recv_slot],
                send_sem=ksend_sems.at[send_slot],
                recv_sem=krecv_sems.at[recv_slot],
                device_id=(right,),
                device_id_type=pl.DeviceIdType.MESH,
            )
            vrdma = pltpu.make_async_remote_copy(
                src_ref=vcomm.at[send_slot],
                dst_ref=vcomm.at[recv_slot],
                send_sem=vsend_sems.at[send_slot],
                recv_sem=vrecv_sems.at[recv_slot],
                device_id=(right,),
                device_id_type=pl.DeviceIdType.MESH,
            )
            krdma.start()
            vrdma.start()
            krdma.wait()
            vrdma.wait()

            origin = (my - h - 1) % N_DEV
            kfull[:, pl.ds(origin * SKV_PER, SKV_PER), :, :] = kcomm[recv_slot]
            vfull[:, pl.ds(origin * SKV_PER, SKV_PER), :, :] = vcomm[recv_slot]

        wq = wq_ref[...].astype(jnp.bfloat16)
        wo = wo_ref[...].astype(jnp.bfloat16)
        qi = lax.broadcasted_iota(jnp.int32, (SQ, SKV), 0)
        ki = lax.broadcasted_iota(jnp.int32, (SQ, SKV), 1)
        mask = jnp.abs(qi - ki) <= WINDOW

        for b in range(B):
            q_b = jnp.dot(x_ref[b].astype(jnp.bfloat16), wq,
                          preferred_element_type=jnp.float32)
            ctx_parts = []
            for h in range(HQ):
                qh = q_b[:, h * DH:(h + 1) * DH].astype(jnp.bfloat16)
                kh = kfull[b, :, h, :].astype(jnp.bfloat16)
                vh = vfull[b, :, h, :].astype(jnp.bfloat16)
                scores = lax.dot_general(
                    qh, kh, (((1,), (1,)), ((), ())),
                    preferred_element_type=jnp.float32,
                ) * 0.125
                scores = jnp.where(mask, scores, -1e9)
                m = jnp.max(scores, axis=1, keepdims=True)
                w = jnp.exp(scores - m)
                w = jnp.where(mask, w, 0.0)
                denom = jnp.sum(w, axis=1, keepdims=True)
                p = (w / denom).astype(jnp.bfloat16)
                ctx_parts.append(jnp.dot(p, vh, preferred_element_type=jnp.float32))
            ctx = jnp.concatenate(ctx_parts, axis=1).astype(jnp.bfloat16)
            out_ref[b] = jnp.dot(ctx, wo, preferred_element_type=jnp.float32)

    return pl.pallas_call(
        body,
        out_shape=jax.ShapeDtypeStruct((B, SQ, HQ * DH * 2), jnp.float32),
        in_specs=[pl.BlockSpec(memory_space=pltpu.VMEM)] * 5,
        out_specs=pl.BlockSpec(memory_space=pltpu.VMEM),
        scratch_shapes=[
            pltpu.VMEM((B, SKV, HQ, DH), jnp.float32),
            pltpu.VMEM((B, SKV, HQ, DH), jnp.float32),
            pltpu.VMEM((2, B, SKV_PER, HQ, DH), jnp.float32),
            pltpu.VMEM((2, B, SKV_PER, HQ, DH), jnp.float32),
            pltpu.SemaphoreType.DMA((2,)),
            pltpu.SemaphoreType.DMA((2,)),
            pltpu.SemaphoreType.DMA((2,)),
            pltpu.SemaphoreType.DMA((2,)),
        ],
        compiler_params=pltpu.CompilerParams(collective_id=0),
    )(x, Wq, K_ext, V_ext, Wo)


# device time: 23521 ns/iter; 4.0208x vs baseline; 4.0208x over previous
import jax
import jax.numpy as jnp
from jax import lax
from jax.experimental import pallas as pl
from jax.experimental.pallas import tpu as pltpu

N_DEV = 4
B = 2
SQ = 256
SKV_PER = 256
HQ = 4
DH = 64
DOUT = 512
NBH = B * HQ


def kernel(x, Wq, K_ext, V_ext, Wo):
    xb = x.astype(jnp.bfloat16)
    wq = Wq.astype(jnp.bfloat16)
    wo = Wo.astype(jnp.bfloat16)
    kt = K_ext.transpose(0, 2, 1, 3).astype(jnp.bfloat16)
    vt = V_ext.transpose(0, 2, 1, 3).astype(jnp.bfloat16)

    def body(x_ref, wq_ref, k_ref, v_ref, wo_ref, out_ref,
             o_own, s_own, recv_o, recv_s,
             o_send_sems, s_send_sems, o_recv_sems, s_recv_sems):
        my = lax.axis_index("i")

        barrier_sem = pltpu.get_barrier_semaphore()
        for p in range(1, N_DEV):
            pl.semaphore_signal(
                barrier_sem, inc=1,
                device_id=((my + p) % N_DEV,),
                device_id_type=pl.DeviceIdType.MESH,
            )

        @pl.when(my == 0)
        def _():
            qi = lax.broadcasted_iota(jnp.int32, (SQ, SKV_PER), 0)
            kj = lax.broadcasted_iota(jnp.int32, (SQ, SKV_PER), 1)
            mask = jnp.abs(qi - kj) <= 128
            for b in range(B):
                qb = jnp.dot(x_ref[b], wq_ref[...],
                             preferred_element_type=jnp.float32)
                for h in range(HQ):
                    qh = (qb[:, h * DH:(h + 1) * DH] * 0.125).astype(jnp.bfloat16)
                    sc = lax.dot_general(
                        qh, k_ref[b, h], (((1,), (1,)), ((), ())),
                        preferred_element_type=jnp.float32)
                    w = jnp.where(mask, jnp.exp(sc), 0.0)
                    idx = b * HQ + h
                    s_own[:, idx:idx + 1] = jnp.sum(w, axis=1, keepdims=True)
                    o_own[b, h] = jnp.dot(
                        w.astype(jnp.bfloat16), v_ref[b, h],
                        preferred_element_type=jnp.float32).astype(jnp.bfloat16)

        @pl.when(my == 1)
        def _():
            o_own[...] = jnp.zeros((B, HQ, SQ, DH), jnp.bfloat16)
            s_own[...] = jnp.zeros((SQ, NBH), jnp.float32)
            r = lax.broadcasted_iota(jnp.int32, (SQ // 2, SQ // 2), 0)
            c = lax.broadcasted_iota(jnp.int32, (SQ // 2, SQ // 2), 1)
            mask = r >= c
            for b in range(B):
                qb = jnp.dot(x_ref[b, SQ // 2:, :], wq_ref[...],
                             preferred_element_type=jnp.float32)
                for h in range(HQ):
                    qh = (qb[:, h * DH:(h + 1) * DH] * 0.125).astype(jnp.bfloat16)
                    sc = lax.dot_general(
                        qh, k_ref[b, h, :SQ // 2, :], (((1,), (1,)), ((), ())),
                        preferred_element_type=jnp.float32)
                    w = jnp.where(mask, jnp.exp(sc), 0.0)
                    idx = b * HQ + h
                    s_own[SQ // 2:, idx:idx + 1] = jnp.sum(w, axis=1, keepdims=True)
                    o_own[b, h, SQ // 2:, :] = jnp.dot(
                        w.astype(jnp.bfloat16), v_ref[b, h, :SQ // 2, :],
                        preferred_element_type=jnp.float32).astype(jnp.bfloat16)

        pl.semaphore_wait(barrier_sem, N_DEV - 1)

        def send_all(slot):
            for d, peer in enumerate([(my + p) % N_DEV for p in range(1, N_DEV)]):
                pltpu.make_async_remote_copy(
                    src_ref=o_own, dst_ref=recv_o.at[slot],
                    send_sem=o_send_sems.at[d], recv_sem=o_recv_sems.at[slot],
                    device_id=(peer,), device_id_type=pl.DeviceIdType.MESH,
                ).start()
                pltpu.make_async_remote_copy(
                    src_ref=s_own, dst_ref=recv_s.at[slot],
                    send_sem=s_send_sems.at[d], recv_sem=s_recv_sems.at[slot],
                    device_id=(peer,), device_id_type=pl.DeviceIdType.MESH,
                ).start()
            recv_o[slot] = o_own[...]
            recv_s[slot] = s_own[...]

        @pl.when(my == 0)
        def _():
            send_all(0)

        @pl.when(my == 1)
        def _():
            send_all(1)

        def wait_slot(slot):
            pltpu.make_async_remote_copy(
                src_ref=recv_o.at[slot], dst_ref=recv_o.at[slot],
                send_sem=o_send_sems.at[0], recv_sem=o_recv_sems.at[slot],
                device_id=(0,), device_id_type=pl.DeviceIdType.MESH,
            ).wait_recv()
            pltpu.make_async_remote_copy(
                src_ref=recv_s.at[slot], dst_ref=recv_s.at[slot],
                send_sem=s_send_sems.at[0], recv_sem=s_recv_sems.at[slot],
                device_id=(0,), device_id_type=pl.DeviceIdType.MESH,
            ).wait_recv()

        @pl.when(my != 0)
        def _():
            wait_slot(0)

        @pl.when(my != 1)
        def _():
            wait_slot(1)

        o_sum = recv_o[0].astype(jnp.float32) + recv_o[1].astype(jnp.float32)
        s_tot = recv_s[0] + recv_s[1]
        for b in range(B):
            acc = jnp.zeros((SQ, DOUT), jnp.float32)
            for h in range(HQ):
                idx = b * HQ + h
                ctx = (o_sum[b, h] / s_tot[:, idx:idx + 1]).astype(jnp.bfloat16)
                acc = acc + jnp.dot(ctx, wo_ref[h * DH:(h + 1) * DH, :],
                                    preferred_element_type=jnp.float32)
            out_ref[b] = acc

        def drain_sends():
            for d in range(N_DEV - 1):
                pltpu.make_async_remote_copy(
                    src_ref=o_own, dst_ref=recv_o.at[0],
                    send_sem=o_send_sems.at[d], recv_sem=o_recv_sems.at[0],
                    device_id=(0,), device_id_type=pl.DeviceIdType.MESH,
                ).wait_send()
                pltpu.make_async_remote_copy(
                    src_ref=s_own, dst_ref=recv_s.at[0],
                    send_sem=s_send_sems.at[d], recv_sem=s_recv_sems.at[0],
                    device_id=(0,), device_id_type=pl.DeviceIdType.MESH,
                ).wait_send()

        @pl.when(my <= 1)
        def _():
            drain_sends()

    return pl.pallas_call(
        body,
        out_shape=jax.ShapeDtypeStruct((B, SQ, DOUT), jnp.float32),
        in_specs=[pl.BlockSpec(memory_space=pltpu.VMEM)] * 5,
        out_specs=pl.BlockSpec(memory_space=pltpu.VMEM),
        scratch_shapes=[
            pltpu.VMEM((B, HQ, SQ, DH), jnp.bfloat16),
            pltpu.VMEM((SQ, NBH), jnp.float32),
            pltpu.VMEM((2, B, HQ, SQ, DH), jnp.bfloat16),
            pltpu.VMEM((2, SQ, NBH), jnp.float32),
            pltpu.SemaphoreType.DMA((N_DEV - 1,)),
            pltpu.SemaphoreType.DMA((N_DEV - 1,)),
            pltpu.SemaphoreType.DMA((2,)),
            pltpu.SemaphoreType.DMA((2,)),
        ],
        compiler_params=pltpu.CompilerParams(collective_id=0),
    )(xb, wq, kt, vt, wo)


# device time: 14422 ns/iter; 6.5576x vs baseline; 1.6309x over previous
import jax
import jax.numpy as jnp
from jax import lax
from jax.experimental import pallas as pl
from jax.experimental.pallas import tpu as pltpu

N_DEV = 4
B = 2
SQ = 256
QR = SQ // 4
SKV_PER = 256
HQ = 4
DH = 64
DOUT = 512
NBH = B * HQ
NSLOT = NBH + 1
NPKT = 6


def kernel(x, Wq, K_ext, V_ext, Wo):
    kr = K_ext.reshape(B, SKV_PER, HQ * DH)
    vr = V_ext.reshape(B, SKV_PER, HQ * DH)

    def body(x_ref, wq_ref, k_ref, v_ref, wo_ref, out_ref,
             o_own, recv, send_sems, recv_sems):
        my = lax.axis_index("i")

        barrier_sem = pltpu.get_barrier_semaphore()
        for p in range(1, N_DEV):
            pl.semaphore_signal(
                barrier_sem, inc=1,
                device_id=((my + p) % N_DEV,),
                device_id_type=pl.DeviceIdType.MESH,
            )

        def rdma(src, slot, peer, d):
            return pltpu.make_async_remote_copy(
                src_ref=src, dst_ref=recv.at[slot],
                send_sem=send_sems.at[d], recv_sem=recv_sems.at[slot],
                device_id=(peer,), device_id_type=pl.DeviceIdType.MESH,
            )

        def wait_packet(slot):
            rdma(recv.at[slot], slot, 0, 0).wait_recv()

        def drain(n):
            for d in range(n):
                rdma(o_own.at[0], 0, 0, d).wait_send()

        @pl.when(my == 0)
        def _():
            wq = wq_ref[...].astype(jnp.bfloat16)
            qi = lax.broadcasted_iota(jnp.int32, (SQ, SKV_PER), 0)
            kj = lax.broadcasted_iota(jnp.int32, (SQ, SKV_PER), 1)
            mask = jnp.abs(qi - kj) <= 128
            for b in range(B):
                qb = jnp.dot(x_ref[b].astype(jnp.bfloat16), wq,
                             preferred_element_type=jnp.float32)
                kv = k_ref[b].astype(jnp.bfloat16)
                vv = v_ref[b].astype(jnp.bfloat16)
                for h in range(HQ):
                    qh = (qb[:, h * DH:(h + 1) * DH] * 0.125).astype(jnp.bfloat16)
                    sc = lax.dot_general(
                        qh, kv[:, h * DH:(h + 1) * DH],
                        (((1,), (1,)), ((), ())),
                        preferred_element_type=jnp.float32)
                    w = jnp.where(mask, jnp.exp(sc), 0.0)
                    s_col = jnp.sum(w, axis=1, keepdims=True).astype(jnp.bfloat16)
                    o_bh = jnp.dot(
                        w.astype(jnp.bfloat16), vv[:, h * DH:(h + 1) * DH],
                        preferred_element_type=jnp.float32).astype(jnp.bfloat16)
                    idx = b * HQ + h
                    for q in range(4):
                        o_own[q, idx] = o_bh[q * QR:(q + 1) * QR]
                        o_own[q, NBH, :, idx:idx + 1] = s_col[q * QR:(q + 1) * QR]

        @pl.when(my == 1)
        def _():
            wq = wq_ref[...].astype(jnp.bfloat16)
            r = lax.broadcasted_iota(jnp.int32, (SQ // 2, SQ // 2), 0)
            c = lax.broadcasted_iota(jnp.int32, (SQ // 2, SQ // 2), 1)
            mask = r >= c
            for b in range(B):
                qb = jnp.dot(x_ref[b, SQ // 2:, :].astype(jnp.bfloat16), wq,
                             preferred_element_type=jnp.float32)
                kv = k_ref[b, :SQ // 2, :].astype(jnp.bfloat16)
                vv = v_ref[b, :SQ // 2, :].astype(jnp.bfloat16)
                for h in range(HQ):
                    qh = (qb[:, h * DH:(h + 1) * DH] * 0.125).astype(jnp.bfloat16)
                    sc = lax.dot_general(
                        qh, kv[:, h * DH:(h + 1) * DH],
                        (((1,), (1,)), ((), ())),
                        preferred_element_type=jnp.float32)
                    w = jnp.where(mask, jnp.exp(sc), 0.0)
                    s_col = jnp.sum(w, axis=1, keepdims=True).astype(jnp.bfloat16)
                    o_bh = jnp.dot(
                        w.astype(jnp.bfloat16), vv[:, h * DH:(h + 1) * DH],
                        preferred_element_type=jnp.float32).astype(jnp.bfloat16)
                    idx = b * HQ + h
                    o_own[2, idx] = o_bh[:QR]
                    o_own[3, idx] = o_bh[QR:]
                    o_own[2, NBH, :, idx:idx + 1] = s_col[:QR]
                    o_own[3, NBH, :, idx:idx + 1] = s_col[QR:]

        @pl.when(my == 0)
        def _():
            pl.semaphore_wait(barrier_sem, N_DEV - 1)
            for d, (q, peer) in enumerate(
                    [(0, 1), (1, 1), (2, 1), (3, 1),
                     (2, 3), (3, 3), (0, 3), (1, 3)]):
                rdma(o_own.at[q], q, peer, d).start()
            for q in range(4):
                recv[q] = o_own[q]
            wait_packet(4)
            wait_packet(5)

        @pl.when(my == 1)
        def _():
            pl.semaphore_wait(barrier_sem, N_DEV - 1)
            rdma(o_own.at[2], 4, 0, 0).start()
            rdma(o_own.at[3], 5, 0, 1).start()
            rdma(o_own.at[2], 4, 2, 2).start()
            rdma(o_own.at[3], 5, 2, 3).start()
            recv[4] = o_own[2]
            recv[5] = o_own[3]
            wait_packet(0)
            rdma(recv.at[0], 0, 2, 4).start()
            wait_packet(1)
            rdma(recv.at[1], 1, 2, 5).start()
            wait_packet(2)
            wait_packet(3)

        @pl.when(my == 2)
        def _():
            pl.semaphore_wait(barrier_sem, N_DEV - 1)
            wait_packet(4)
            rdma(recv.at[4], 4, 3, 0).start()
            wait_packet(5)
            rdma(recv.at[5], 5, 3, 1).start()
            for q in range(4):
                wait_packet(q)

        @pl.when(my == 3)
        def _():
            pl.semaphore_wait(barrier_sem, N_DEV - 1)
            wait_packet(2)
            rdma(recv.at[2], 2, 2, 0).start()
            wait_packet(3)
            rdma(recv.at[3], 3, 2, 1).start()
            wait_packet(0)
            wait_packet(1)
            wait_packet(4)
            wait_packet(5)

        m = [recv[0].astype(jnp.float32),
             recv[1].astype(jnp.float32),
             recv[2].astype(jnp.float32) + recv[4].astype(jnp.float32),
             recv[3].astype(jnp.float32) + recv[5].astype(jnp.float32)]
        wo = wo_ref[...].astype(jnp.bfloat16)
        for b in range(B):
            acc = jnp.zeros((SQ, DOUT), jnp.float32)
            for h in range(HQ):
                idx = b * HQ + h
                ctx = jnp.concatenate(
                    [m[q][idx] / m[q][NBH, :, idx:idx + 1] for q in range(4)],
                    axis=0).astype(jnp.bfloat16)
                acc = acc + jnp.dot(ctx, wo[h * DH:(h + 1) * DH, :],
                                    preferred_element_type=jnp.float32)
            out_ref[b] = acc

        @pl.when(my == 0)
        def _():
            drain(8)

        @pl.when(my == 1)
        def _():
            drain(6)

        @pl.when(my >= 2)
        def _():
            drain(2)

    return pl.pallas_call(
        body,
        out_shape=jax.ShapeDtypeStruct((B, SQ, DOUT), jnp.float32),
        in_specs=[pl.BlockSpec(memory_space=pltpu.VMEM)] * 5,
        out_specs=pl.BlockSpec(memory_space=pltpu.VMEM),
        scratch_shapes=[
            pltpu.VMEM((4, NSLOT, QR, DH), jnp.bfloat16),
            pltpu.VMEM((NPKT, NSLOT, QR, DH), jnp.bfloat16),
            pltpu.SemaphoreType.DMA((8,)),
            pltpu.SemaphoreType.DMA((NPKT,)),
        ],
        compiler_params=pltpu.CompilerParams(collective_id=0),
    )(x, Wq, kr, vr, Wo)
